# baseline (speedup 1.0000x reference)
import functools

import jax
import jax.numpy as jnp
from jax.experimental import pallas as pl
from jax.experimental.pallas import tpu as pltpu

TM = 512
HB = 512


def _ffn_body(pt_ref, ex_ref, rs_ref, re_ref, first_ref,
              xs_ref, w1_ref, w3_ref, w2_ref, out_ref, acc_ref, *, ht):
    h = pl.program_id(1)

    @pl.when(h == 0)
    def _():
        acc_ref[...] = jnp.zeros_like(acc_ref)

    x = xs_ref[...]
    g = jnp.dot(x, w1_ref[0], preferred_element_type=jnp.float32)
    u = jnp.dot(x, w3_ref[0], preferred_element_type=jnp.float32)
    mid = (g * jax.nn.sigmoid(g) * u).astype(jnp.bfloat16)
    acc_ref[...] += jnp.dot(mid, w2_ref[0], preferred_element_type=jnp.float32)

    @pl.when(h == ht - 1)
    def _():
        i = pl.program_id(0)
        rs = rs_ref[i]
        re = re_ref[i]
        first = first_ref[i]
        rows = jax.lax.broadcasted_iota(jnp.int32, out_ref.shape, 0)
        mask = (rows >= rs) & (rows < re)
        prev = jnp.where(first == 1, jnp.zeros_like(out_ref), out_ref[...])
        out_ref[...] = jnp.where(mask, acc_ref[...], prev)


def _grouped_ffn(xs, W1, W3, W2, pt, ex, rs, re, first, interpret=False):
    Ts, D = xs.shape
    E, _, H = W1.shape
    L = pt.shape[0]
    ht = H // HB

    xs = xs.astype(jnp.bfloat16)
    W1 = W1.astype(jnp.bfloat16)
    W3 = W3.astype(jnp.bfloat16)
    W2 = W2.astype(jnp.bfloat16)
    grid_spec = pltpu.PrefetchScalarGridSpec(
        num_scalar_prefetch=5,
        grid=(L, ht),
        in_specs=[
            pl.BlockSpec((TM, D), lambda i, h, pt, ex, rs, re, fi: (pt[i], 0)),
            pl.BlockSpec((1, D, HB), lambda i, h, pt, ex, rs, re, fi: (ex[i], 0, h)),
            pl.BlockSpec((1, D, HB), lambda i, h, pt, ex, rs, re, fi: (ex[i], 0, h)),
            pl.BlockSpec((1, HB, D), lambda i, h, pt, ex, rs, re, fi: (ex[i], h, 0)),
        ],
        out_specs=pl.BlockSpec((TM, D), lambda i, h, pt, ex, rs, re, fi: (pt[i], 0)),
        scratch_shapes=[pltpu.VMEM((TM, D), jnp.float32)],
    )
    return pl.pallas_call(
        functools.partial(_ffn_body, ht=ht),
        grid_spec=grid_spec,
        out_shape=jax.ShapeDtypeStruct((Ts, D), jnp.float32),
        compiler_params=pltpu.CompilerParams(
            dimension_semantics=("arbitrary", "arbitrary"),
        ),
        interpret=pltpu.InterpretParams() if interpret else False,
    )(pt, ex, rs, re, first, xs, W1, W3, W2)


def _tile_metadata(starts, ends, num_tiles, L):
    E = starts.shape[0]
    m = jnp.arange(num_tiles, dtype=jnp.int32)[:, None]
    lo, hi = m * TM, (m + 1) * TM
    st = starts[None, :].astype(jnp.int32)
    en = ends[None, :].astype(jnp.int32)
    act = (st < hi) & (en > lo)
    rs = jnp.clip(st - lo, 0, TM)
    re = jnp.clip(en - lo, 0, TM)
    ex = jnp.broadcast_to(jnp.arange(E, dtype=jnp.int32)[None, :], act.shape)
    pt = jnp.broadcast_to(m, act.shape)

    actf = act.reshape(-1)
    pos = jnp.where(actf, jnp.cumsum(actf) - 1, L + 1)
    n_real = jnp.sum(actf.astype(jnp.int32))

    def place(v):
        a = jnp.zeros((L,), jnp.int32).at[pos].set(
            v.reshape(-1).astype(jnp.int32), mode="drop")
        sel = jnp.minimum(jnp.arange(L), n_real - 1)
        return a[sel]

    pt_a, ex_a, rs_a, re_a = place(pt), place(ex), place(rs), place(re)
    first_a = (rs_a == 0).astype(jnp.int32)
    return pt_a, ex_a, rs_a, re_a, first_a


def kernel(x, Wr, W1, W3, W2, interpret=False):
    b, s, d = x.shape
    xf = x.reshape(-1, d)
    T = xf.shape[0]
    E = Wr.shape[1]
    K = 2

    logits = xf @ Wr
    probs = jax.nn.softmax(logits, axis=-1)
    topw, topi = jax.lax.top_k(probs, K)
    wts = topw / jnp.sum(topw, axis=-1, keepdims=True)

    counts = jnp.bincount(topi.reshape(-1), length=E)
    aux_loss = E * jnp.sum((counts.astype(jnp.float32) / (T * K))
                           * probs.mean(axis=0))

    Ts = T * K
    e_flat = topi.reshape(-1)
    sort_idx = jnp.argsort(e_flat, stable=True)
    tok_ids = (sort_idx // K).astype(jnp.int32)
    w_sorted = wts.reshape(-1)[sort_idx]
    inv = jnp.zeros((Ts,), jnp.int32).at[sort_idx].set(
        jnp.arange(Ts, dtype=jnp.int32))
    inv = inv.reshape(T, K)

    starts = jnp.cumsum(counts) - counts
    ends = starts + counts
    M = Ts // TM
    L = M + E - 1
    pt, ex, rs, re, first = _tile_metadata(starts, ends, M, L)

    xs = jnp.take(xf, tok_ids, axis=0)
    ys = _grouped_ffn(xs, W1, W3, W2, pt, ex, rs, re, first,
                      interpret=interpret)
    yw = ys * w_sorted[:, None]
    out = yw[inv[:, 0]] + yw[inv[:, 1]]
    return out.reshape(b, s, d), aux_loss


# speedup vs baseline: 1.2340x; 1.2340x over previous
import functools

import jax
import jax.numpy as jnp
from jax.experimental import pallas as pl
from jax.experimental.pallas import tpu as pltpu

TM = 512
HB = 512


def _ffn_body(pt_ref, ex_ref, rs_ref, re_ref, first_ref,
              xs_ref, w1_ref, w3_ref, w2_ref, out_ref, acc_ref, *, ht):
    h = pl.program_id(1)

    @pl.when(h == 0)
    def _():
        acc_ref[...] = jnp.zeros_like(acc_ref)

    x = xs_ref[...].astype(jnp.bfloat16)
    g = jnp.dot(x, w1_ref[0].astype(jnp.bfloat16),
                preferred_element_type=jnp.float32)
    u = jnp.dot(x, w3_ref[0].astype(jnp.bfloat16),
                preferred_element_type=jnp.float32)
    mid = (g * jax.nn.sigmoid(g) * u).astype(jnp.bfloat16)
    acc_ref[...] += jnp.dot(mid, w2_ref[0].astype(jnp.bfloat16),
                            preferred_element_type=jnp.float32)

    @pl.when(h == ht - 1)
    def _():
        i = pl.program_id(0)
        rs = rs_ref[i]
        re = re_ref[i]
        first = first_ref[i]
        rows = jax.lax.broadcasted_iota(jnp.int32, out_ref.shape, 0)
        mask = (rows >= rs) & (rows < re)
        prev = jnp.where(first == 1, jnp.zeros_like(out_ref), out_ref[...])
        out_ref[...] = jnp.where(mask, acc_ref[...], prev)


def _grouped_ffn(xs, W1, W3, W2, pt, ex, rs, re, first, interpret=False):
    Ts, D = xs.shape
    E, _, H = W1.shape
    L = pt.shape[0]
    ht = H // HB

    grid_spec = pltpu.PrefetchScalarGridSpec(
        num_scalar_prefetch=5,
        grid=(L, ht),
        in_specs=[
            pl.BlockSpec((TM, D), lambda i, h, pt, ex, rs, re, fi: (pt[i], 0)),
            pl.BlockSpec((1, D, HB), lambda i, h, pt, ex, rs, re, fi: (ex[i], 0, h)),
            pl.BlockSpec((1, D, HB), lambda i, h, pt, ex, rs, re, fi: (ex[i], 0, h)),
            pl.BlockSpec((1, HB, D), lambda i, h, pt, ex, rs, re, fi: (ex[i], h, 0)),
        ],
        out_specs=pl.BlockSpec((TM, D), lambda i, h, pt, ex, rs, re, fi: (pt[i], 0)),
        scratch_shapes=[pltpu.VMEM((TM, D), jnp.float32)],
    )
    return pl.pallas_call(
        functools.partial(_ffn_body, ht=ht),
        grid_spec=grid_spec,
        out_shape=jax.ShapeDtypeStruct((Ts, D), jnp.float32),
        compiler_params=pltpu.CompilerParams(
            dimension_semantics=("arbitrary", "arbitrary"),
        ),
        interpret=pltpu.InterpretParams() if interpret else False,
    )(pt, ex, rs, re, first, xs, W1, W3, W2)


def _tile_metadata(starts, ends, num_tiles, L):
    E = starts.shape[0]
    m = jnp.arange(num_tiles, dtype=jnp.int32)[:, None]
    lo, hi = m * TM, (m + 1) * TM
    st = starts[None, :].astype(jnp.int32)
    en = ends[None, :].astype(jnp.int32)
    act = (st < hi) & (en > lo)
    rs = jnp.clip(st - lo, 0, TM)
    re = jnp.clip(en - lo, 0, TM)
    ex = jnp.broadcast_to(jnp.arange(E, dtype=jnp.int32)[None, :], act.shape)
    pt = jnp.broadcast_to(m, act.shape)

    actf = act.reshape(-1)
    pos = jnp.where(actf, jnp.cumsum(actf) - 1, L + 1)
    n_real = jnp.sum(actf.astype(jnp.int32))

    def place(v):
        a = jnp.zeros((L,), jnp.int32).at[pos].set(
            v.reshape(-1).astype(jnp.int32), mode="drop")
        sel = jnp.minimum(jnp.arange(L), n_real - 1)
        return a[sel]

    pt_a, ex_a, rs_a, re_a = place(pt), place(ex), place(rs), place(re)
    first_a = (rs_a == 0).astype(jnp.int32)
    return pt_a, ex_a, rs_a, re_a, first_a


def kernel(x, Wr, W1, W3, W2, interpret=False):
    b, s, d = x.shape
    xf = x.reshape(-1, d)
    T = xf.shape[0]
    E = Wr.shape[1]
    K = 2

    logits = xf @ Wr
    probs = jax.nn.softmax(logits, axis=-1)
    topw, topi = jax.lax.top_k(probs, K)
    wts = topw / jnp.sum(topw, axis=-1, keepdims=True)

    counts = jnp.bincount(topi.reshape(-1), length=E)
    aux_loss = E * jnp.sum((counts.astype(jnp.float32) / (T * K))
                           * probs.mean(axis=0))

    Ts = T * K
    e_flat = topi.reshape(-1)
    sort_idx = jnp.argsort(e_flat, stable=True)
    tok_ids = (sort_idx // K).astype(jnp.int32)
    w_sorted = wts.reshape(-1)[sort_idx]
    inv = jnp.zeros((Ts,), jnp.int32).at[sort_idx].set(
        jnp.arange(Ts, dtype=jnp.int32))
    inv = inv.reshape(T, K)

    starts = jnp.cumsum(counts) - counts
    ends = starts + counts
    M = Ts // TM
    L = M + E - 1
    pt, ex, rs, re, first = _tile_metadata(starts, ends, M, L)

    xs = jnp.take(xf, tok_ids, axis=0)
    ys = _grouped_ffn(xs, W1, W3, W2, pt, ex, rs, re, first,
                      interpret=interpret)
    yw = ys * w_sorted[:, None]
    out = yw[inv[:, 0]] + yw[inv[:, 1]]
    return out.reshape(b, s, d), aux_loss


# speedup vs baseline: 1.2430x; 1.0073x over previous
import functools

import jax
import jax.numpy as jnp
from jax.experimental import pallas as pl
from jax.experimental.pallas import tpu as pltpu

TM = 1024
HB = 256


def _ffn_body(pt_ref, ex_ref, rs_ref, re_ref, first_ref,
              xs_ref, w1_ref, w3_ref, w2_ref, out_ref, acc_ref, *, ht):
    h = pl.program_id(1)

    @pl.when(h == 0)
    def _():
        acc_ref[...] = jnp.zeros_like(acc_ref)

    x = xs_ref[...]
    g = jnp.dot(x, w1_ref[0], preferred_element_type=jnp.float32)
    u = jnp.dot(x, w3_ref[0], preferred_element_type=jnp.float32)
    mid = g * jax.nn.sigmoid(g) * u
    acc_ref[...] += jnp.dot(mid, w2_ref[0], preferred_element_type=jnp.float32)

    @pl.when(h == ht - 1)
    def _():
        i = pl.program_id(0)
        rs = rs_ref[i]
        re = re_ref[i]
        first = first_ref[i]
        rows = jax.lax.broadcasted_iota(jnp.int32, out_ref.shape, 0)
        mask = (rows >= rs) & (rows < re)
        prev = jnp.where(first == 1, jnp.zeros_like(out_ref), out_ref[...])
        out_ref[...] = jnp.where(mask, acc_ref[...], prev)


def _grouped_ffn(xs, W1, W3, W2, pt, ex, rs, re, first, interpret=False):
    Ts, D = xs.shape
    E, _, H = W1.shape
    L = pt.shape[0]
    ht = H // HB

    grid_spec = pltpu.PrefetchScalarGridSpec(
        num_scalar_prefetch=5,
        grid=(L, ht),
        in_specs=[
            pl.BlockSpec((TM, D), lambda i, h, pt, ex, rs, re, fi: (pt[i], 0)),
            pl.BlockSpec((1, D, HB), lambda i, h, pt, ex, rs, re, fi: (ex[i], 0, h)),
            pl.BlockSpec((1, D, HB), lambda i, h, pt, ex, rs, re, fi: (ex[i], 0, h)),
            pl.BlockSpec((1, HB, D), lambda i, h, pt, ex, rs, re, fi: (ex[i], h, 0)),
        ],
        out_specs=pl.BlockSpec((TM, D), lambda i, h, pt, ex, rs, re, fi: (pt[i], 0)),
        scratch_shapes=[pltpu.VMEM((TM, D), jnp.float32)],
    )
    return pl.pallas_call(
        functools.partial(_ffn_body, ht=ht),
        grid_spec=grid_spec,
        out_shape=jax.ShapeDtypeStruct((Ts, D), jnp.float32),
        compiler_params=pltpu.CompilerParams(
            dimension_semantics=("arbitrary", "arbitrary"),
        ),
        interpret=pltpu.InterpretParams() if interpret else False,
    )(pt, ex, rs, re, first, xs, W1, W3, W2)


def _tile_metadata(starts, ends, num_tiles, L):
    E = starts.shape[0]
    m = jnp.arange(num_tiles, dtype=jnp.int32)[:, None]
    lo, hi = m * TM, (m + 1) * TM
    st = starts[None, :].astype(jnp.int32)
    en = ends[None, :].astype(jnp.int32)
    act = (st < hi) & (en > lo)
    rs = jnp.clip(st - lo, 0, TM)
    re = jnp.clip(en - lo, 0, TM)
    ex = jnp.broadcast_to(jnp.arange(E, dtype=jnp.int32)[None, :], act.shape)
    pt = jnp.broadcast_to(m, act.shape)

    actf = act.reshape(-1)
    pos = jnp.where(actf, jnp.cumsum(actf) - 1, L + 1)
    n_real = jnp.sum(actf.astype(jnp.int32))

    def place(v):
        a = jnp.zeros((L,), jnp.int32).at[pos].set(
            v.reshape(-1).astype(jnp.int32), mode="drop")
        sel = jnp.minimum(jnp.arange(L), n_real - 1)
        return a[sel]

    pt_a, ex_a, rs_a, re_a = place(pt), place(ex), place(rs), place(re)
    first_a = (rs_a == 0).astype(jnp.int32)
    return pt_a, ex_a, rs_a, re_a, first_a


def kernel(x, Wr, W1, W3, W2, interpret=False):
    b, s, d = x.shape
    xf = x.reshape(-1, d)
    T = xf.shape[0]
    E = Wr.shape[1]
    K = 2

    logits = xf @ Wr
    probs = jax.nn.softmax(logits, axis=-1)
    topw, topi = jax.lax.top_k(probs, K)
    wts = topw / jnp.sum(topw, axis=-1, keepdims=True)

    counts = jnp.bincount(topi.reshape(-1), length=E)
    aux_loss = E * jnp.sum((counts.astype(jnp.float32) / (T * K))
                           * probs.mean(axis=0))

    Ts = T * K
    e_flat = topi.reshape(-1)
    sort_idx = jnp.argsort(e_flat, stable=True)
    tok_ids = (sort_idx // K).astype(jnp.int32)
    w_sorted = wts.reshape(-1)[sort_idx]
    inv = jnp.zeros((Ts,), jnp.int32).at[sort_idx].set(
        jnp.arange(Ts, dtype=jnp.int32))
    inv = inv.reshape(T, K)

    starts = jnp.cumsum(counts) - counts
    ends = starts + counts
    M = Ts // TM
    L = M + E - 1
    pt, ex, rs, re, first = _tile_metadata(starts, ends, M, L)

    xs = jnp.take(xf, tok_ids, axis=0)
    ys = _grouped_ffn(xs, W1, W3, W2, pt, ex, rs, re, first,
                      interpret=interpret)
    yw = ys * w_sorted[:, None]
    out = yw[inv[:, 0]] + yw[inv[:, 1]]
    return out.reshape(b, s, d), aux_loss
